# Initial kernel scaffold; baseline (speedup 1.0000x reference)
import jax
import jax.numpy as jnp
from jax.experimental import pallas as pl


def kernel(codes, ages, visit_ids, code_table, age_table, visit_table, pos_table, ln_w, ln_b):
    raise NotImplementedError("write your pallas kernel here")


# speedup vs baseline: 2.2209x; 2.2209x over previous
import jax
import jax.numpy as jnp
from jax import lax
from jax.experimental import pallas as pl
from jax.experimental.pallas import tpu as pltpu
from jax.experimental.pallas import tpu_sc as plsc

B, S, D = 4096, 200, 64
NUM_AGE_BINS = 22
MAX_VISITS = 512
MAX_POS = 512
LN_EPS = 1e-5

NW = 32
SEQ_PER_W = B // NW
HALF = S // 2


def _rsqrt16(v):
    i = plsc.bitcast(v, jnp.int32)
    i = jnp.int32(0x5F3759DF) - (i >> 1)
    y = plsc.bitcast(i, jnp.float32)
    for _ in range(3):
        y = y * (1.5 - 0.5 * v * y * y)
    return y


def _sc_body(codes_hbm, ages_hbm, vis_hbm, ctab_hbm, atab_hbm, vtab_hbm,
             ptab_hbm, lnw_hbm, lnb_hbm, out_hbm,
             cidx_v, ages_v, vis_v, crows_v, out_v,
             atab_v, vtab_v, ptab_v, lnw_v, lnb_v, gsem):
    wid = lax.axis_index("s") * 2 + lax.axis_index("c")

    pltpu.sync_copy(atab_hbm, atab_v)
    pltpu.sync_copy(vtab_hbm, vtab_v)
    pltpu.sync_copy(ptab_hbm.at[pl.ds(0, S)], ptab_v)
    pltpu.sync_copy(lnw_hbm, lnw_v)
    pltpu.sync_copy(lnb_hbm, lnb_v)

    iot = lax.iota(jnp.int32, 16)
    w_regs = [lnw_v[pl.ds(16 * k, 16)] for k in range(4)]
    b_regs = [lnb_v[pl.ds(16 * k, 16)] for k in range(4)]

    def seq_body(i, carry):
        b = wid * SEQ_PER_W + i
        pltpu.sync_copy(codes_hbm.at[b], cidx_v)
        pltpu.sync_copy(ages_hbm.at[b], ages_v)
        pltpu.sync_copy(vis_hbm.at[b], vis_v)
        cp0 = pltpu.async_copy(ctab_hbm.at[cidx_v.at[0]],
                               crows_v.at[pl.ds(0, HALF)], gsem)
        cp1 = pltpu.async_copy(ctab_hbm.at[cidx_v.at[1]],
                               crows_v.at[pl.ds(HALF, HALF)], gsem)
        cp0.wait()
        cp1.wait()

        @plsc.parallel_loop(0, 13, step=1)
        def tok_group(g):
            t0 = jnp.minimum(g * 16, S - 16)
            age16 = jnp.maximum(ages_v[pl.ds(t0, 16)], 0)
            abin16 = lax.div(jnp.minimum(age16, 100), 5)
            vrow16 = jnp.minimum(jnp.maximum(vis_v[pl.ds(t0, 16)], 0),
                                 MAX_VISITS - 1)
            for j in range(16):
                t = t0 + j
                rowa = jnp.full((16,), abin16[j], jnp.int32)
                rowv = jnp.full((16,), vrow16[j], jnp.int32)
                x = []
                for k in range(4):
                    col = iot + (16 * k)
                    xc = crows_v[t, pl.ds(16 * k, 16)]
                    xa = plsc.load_gather(atab_v, [rowa, col])
                    xv = plsc.load_gather(vtab_v, [rowv, col])
                    xp = ptab_v[t, pl.ds(16 * k, 16)]
                    x.append((xc + xa) + (xv + xp))
                tot = jnp.sum((x[0] + x[1]) + (x[2] + x[3]))
                ssq = jnp.sum((x[0] * x[0] + x[1] * x[1])
                              + (x[2] * x[2] + x[3] * x[3]))
                mean = tot * (1.0 / 64.0)
                var = ssq * (1.0 / 64.0) - mean * mean
                rstd = _rsqrt16(jnp.full((16,), var + LN_EPS, jnp.float32))
                for k in range(4):
                    out_v[t, pl.ds(16 * k, 16)] = (
                        (x[k] - mean) * rstd * w_regs[k] + b_regs[k])

        pltpu.sync_copy(out_v, out_hbm.at[b])
        return carry

    lax.fori_loop(0, SEQ_PER_W, seq_body, 0)


def kernel(codes, ages, visit_ids, code_table, age_table, visit_table,
           pos_table, ln_w, ln_b):
    codes_r = codes.astype(jnp.int32).reshape(B, 2, HALF)
    ages = ages.astype(jnp.int32)
    vis = visit_ids.astype(jnp.int32)
    mesh = plsc.VectorSubcoreMesh(core_axis_name="c", subcore_axis_name="s")
    f = pl.kernel(
        _sc_body,
        out_type=jax.ShapeDtypeStruct((B, S, D), jnp.float32),
        mesh=mesh,
        compiler_params=pltpu.CompilerParams(needs_layout_passes=False,
                                             use_tc_tiling_on_sc=False),
        scratch_types=[
            pltpu.VMEM((2, HALF), jnp.int32),
            pltpu.VMEM((S,), jnp.int32),
            pltpu.VMEM((S,), jnp.int32),
            pltpu.VMEM((S, D), jnp.float32),
            pltpu.VMEM((S, D), jnp.float32),
            pltpu.VMEM((NUM_AGE_BINS, D), jnp.float32),
            pltpu.VMEM((MAX_VISITS, D), jnp.float32),
            pltpu.VMEM((S, D), jnp.float32),
            pltpu.VMEM((D,), jnp.float32),
            pltpu.VMEM((D,), jnp.float32),
            pltpu.SemaphoreType.DMA,
        ],
    )
    return f(codes_r, ages, vis, code_table, age_table, visit_table,
             pos_table, ln_w, ln_b)
